# baseline (speedup 1.0000x reference)
import functools

import jax
import jax.numpy as jnp
from jax import lax
from jax.experimental import pallas as pl
from jax.experimental.pallas import tpu as pltpu
from jax.experimental.pallas import tpu_sc as plsc

NC = 2
NS = 16
NW = NC * NS
CH = 128


def _emb_kernel(B, D, n_chunks):
    mesh = plsc.VectorSubcoreMesh(core_axis_name="c", subcore_axis_name="s")

    @functools.partial(
        pl.kernel,
        mesh=mesh,
        compiler_params=pltpu.CompilerParams(use_tc_tiling_on_sc=False),
        out_type=jax.ShapeDtypeStruct((B, D), jnp.float32),
        scratch_types=[
            pltpu.VMEM((n_chunks, CH), jnp.int32),
            pltpu.VMEM((CH, D), jnp.float32),
            pltpu.SemaphoreType.DMA,
        ],
    )
    def k(x_hbm, table_hbm, out_hbm, idx_v, rows, gsem):
        wid = lax.axis_index("s") * NC + lax.axis_index("c")
        base = wid * (n_chunks * CH)
        pltpu.sync_copy(x_hbm.at[wid], idx_v)

        def chunk_body(j, _):
            pltpu.async_copy(table_hbm.at[idx_v.at[j]], rows, gsem).wait()

            def scale_row(r, _):
                for c in range(D // 16):
                    sl = pl.ds(c * 16, 16)
                    rows[r, sl] = rows[r, sl] * 8.0
                return 0

            lax.fori_loop(0, CH, scale_row, 0, unroll=4)
            pltpu.sync_copy(rows, out_hbm.at[pl.ds(base + j * CH, CH)])
            return 0

        lax.fori_loop(0, n_chunks, chunk_body, 0)

    return k


def kernel(x, table):
    S0, S1 = x.shape
    V, D = table.shape
    B = S0 * S1
    n_chunks = B // (NW * CH)
    idx = x.reshape(NW, n_chunks, CH).astype(jnp.int32)
    out = _emb_kernel(B, D, n_chunks)(idx, table)
    return out.reshape(S0, S1, D)
